# Initial kernel scaffold; baseline (speedup 1.0000x reference)
import jax
import jax.numpy as jnp
from jax.experimental import pallas as pl


def kernel(x, edge_index, edge_type, W0, W1, W2, Ws, bs):
    raise NotImplementedError("write your pallas kernel here")


# speedup vs baseline: 5.6865x; 5.6865x over previous
import functools

import jax
import jax.numpy as jnp
from jax import lax
from jax.experimental import pallas as pl
from jax.experimental.pallas import tpu as pltpu
from jax.experimental.pallas import tpu_sc as plsc

N_NODES = 10000
N_EDGES = 320000
D = 128

NC = 2
NS = 16
NW = NC * NS
L = 16

CHUNK = 64
EPT = 10240
NCHUNK = EPT // CHUNK
E_PAD = EPT * NW
ACC_ROWS = 10240
RPT = ACC_ROWS // NS
ZR = 16

MM_BLK = 1000


def _transform_body(x_ref, w_ref, b_ref, o_ref):
    o_ref[...] = (
        jnp.dot(x_ref[...], w_ref[...], preferred_element_type=jnp.float32)
        + b_ref[...]
    )


_transform = pl.pallas_call(
    _transform_body,
    grid=(N_NODES // MM_BLK,),
    in_specs=[
        pl.BlockSpec((MM_BLK, D), lambda i: (i, 0)),
        pl.BlockSpec((D, 4 * D), lambda i: (0, 0)),
        pl.BlockSpec((1, 4 * D), lambda i: (0, 0)),
    ],
    out_specs=pl.BlockSpec((MM_BLK, 4 * D), lambda i: (i, 0)),
    out_shape=jax.ShapeDtypeStruct((N_NODES, 4 * D), jnp.float32),
)


def _combine_body(t_ref, p_ref, o_ref):
    o_ref[...] = jnp.maximum(t_ref[...] + p_ref[0] + p_ref[1], 0.0)


_combine = pl.pallas_call(
    _combine_body,
    grid=(N_NODES // MM_BLK,),
    in_specs=[
        pl.BlockSpec((MM_BLK, D), lambda i: (i, 3)),
        pl.BlockSpec((NC, MM_BLK, D), lambda i: (0, i, 0)),
    ],
    out_specs=pl.BlockSpec((MM_BLK, D), lambda i: (i, 0)),
    out_shape=jax.ShapeDtypeStruct((N_NODES, D), jnp.float32),
)


_sc_mesh = plsc.VectorSubcoreMesh(core_axis_name="c", subcore_axis_name="s")


@functools.partial(
    pl.kernel,
    out_type=jax.ShapeDtypeStruct((NC, ACC_ROWS, D), jnp.float32),
    mesh=_sc_mesh,
    scratch_types=[
        pltpu.VMEM((EPT,), jnp.int32),
        pltpu.VMEM((EPT,), jnp.int32),
        pltpu.VMEM((EPT,), jnp.int32),
        pltpu.VMEM((CHUNK,), jnp.int32),
        pltpu.VMEM((CHUNK,), jnp.int32),
        pltpu.VMEM((CHUNK, D), jnp.float32),
        pltpu.VMEM((ZR, D), jnp.float32),
        pltpu.VMEM_SHARED((ACC_ROWS, D), jnp.float32),
        pltpu.SemaphoreType.DMA,
    ],
)
def _edge_scatter(table_hbm, src_hbm, et_hbm, dst_hbm, out_hbm,
                  src_v, et_v, dst_v, g_v, d_v, rows_v, z_v, acc, sem):
    cid = lax.axis_index("c")
    sid = lax.axis_index("s")
    wid = cid * NS + sid

    @pl.loop(0, ZR)
    def _(r):
        @pl.loop(0, D // L)
        def _(c):
            z_v.at[pl.ds(r, 1), pl.ds(c * L, L)][...] = jnp.zeros(
                (1, L), jnp.float32)

    @pl.loop(0, RPT // ZR)
    def _(j):
        pltpu.sync_copy(z_v, acc.at[pl.ds(sid * RPT + j * ZR, ZR)])

    base = wid * EPT
    pltpu.sync_copy(src_hbm.at[pl.ds(base, EPT)], src_v)
    pltpu.sync_copy(et_hbm.at[pl.ds(base, EPT)], et_v)
    pltpu.sync_copy(dst_hbm.at[pl.ds(base, EPT)], dst_v)

    plsc.subcore_barrier()

    @pl.loop(0, NCHUNK)
    def _(i):
        off = i * CHUNK

        @pl.loop(0, CHUNK // L)
        def _(j):
            s_in = pl.ds(off + j * L, L)
            s_out = pl.ds(j * L, L)
            g_v.at[s_out][...] = src_v.at[s_in][...] * 4 + et_v.at[s_in][...]
            d_v.at[s_out][...] = dst_v.at[s_in][...]

        pltpu.async_copy(table_hbm.at[g_v], rows_v, sem).wait()
        pltpu.sync_copy(rows_v, acc.at[d_v], add=True)

    plsc.subcore_barrier()

    pltpu.sync_copy(acc.at[pl.ds(sid * RPT, RPT)],
                    out_hbm.at[cid, pl.ds(sid * RPT, RPT)])


def kernel(x, edge_index, edge_type, W0, W1, W2, Ws, bs):
    x = x.astype(jnp.float32)
    src = edge_index[0].astype(jnp.int32)
    dst = edge_index[1].astype(jnp.int32)
    et = edge_type.astype(jnp.int32)

    pad = E_PAD - N_EDGES
    src = jnp.pad(src, (0, pad))
    et = jnp.pad(et, (0, pad))
    dst = jnp.pad(dst, (0, pad), constant_values=N_NODES)

    w_cat = jnp.concatenate([W0, W1, W2, Ws], axis=0).T
    b_cat = jnp.zeros((1, 4 * D), jnp.float32).at[0, 3 * D:].set(bs)

    table = _transform(x, w_cat, b_cat)
    partials = _edge_scatter(table.reshape(4 * N_NODES, D), src, et, dst)
    return _combine(table, partials)


# speedup vs baseline: 6.8547x; 1.2054x over previous
import functools

import jax
import jax.numpy as jnp
from jax import lax
from jax.experimental import pallas as pl
from jax.experimental.pallas import tpu as pltpu
from jax.experimental.pallas import tpu_sc as plsc

N_NODES = 10000
N_EDGES = 320000
D = 128

NC = 2
NS = 16
NW = NC * NS
L = 16

CHUNK = 64
EPT = 10240
NCHUNK = EPT // CHUNK
E_PAD = EPT * NW
ACC_ROWS = 10112
RPT = ACC_ROWS // NS
ZR = 8

MM_BLK = 1000


def _transform_body(x_ref, w_ref, b_ref, o_ref):
    o_ref[...] = (
        jnp.dot(x_ref[...], w_ref[...], preferred_element_type=jnp.float32)
        + b_ref[...]
    )


_transform = pl.pallas_call(
    _transform_body,
    grid=(N_NODES // MM_BLK,),
    in_specs=[
        pl.BlockSpec((MM_BLK, D), lambda i: (i, 0)),
        pl.BlockSpec((D, 4 * D), lambda i: (0, 0)),
        pl.BlockSpec((1, 4 * D), lambda i: (0, 0)),
    ],
    out_specs=pl.BlockSpec((MM_BLK, 4 * D), lambda i: (i, 0)),
    out_shape=jax.ShapeDtypeStruct((N_NODES, 4 * D), jnp.float32),
)


def _combine_body(t_ref, p_ref, o_ref):
    o_ref[...] = jnp.maximum(t_ref[...] + p_ref[0] + p_ref[1], 0.0)


_combine = pl.pallas_call(
    _combine_body,
    grid=(N_NODES // MM_BLK,),
    in_specs=[
        pl.BlockSpec((MM_BLK, D), lambda i: (i, 3)),
        pl.BlockSpec((NC, MM_BLK, D), lambda i: (0, i, 0)),
    ],
    out_specs=pl.BlockSpec((MM_BLK, D), lambda i: (i, 0)),
    out_shape=jax.ShapeDtypeStruct((N_NODES, D), jnp.float32),
)


_sc_mesh = plsc.VectorSubcoreMesh(core_axis_name="c", subcore_axis_name="s")


@functools.partial(
    pl.kernel,
    out_type=jax.ShapeDtypeStruct((NC, ACC_ROWS, D), jnp.float32),
    mesh=_sc_mesh,
    scratch_types=[
        pltpu.VMEM((EPT,), jnp.int32),
        pltpu.VMEM((EPT,), jnp.int32),
        pltpu.VMEM((EPT,), jnp.int32),
        pltpu.VMEM((CHUNK,), jnp.int32),
        pltpu.VMEM((CHUNK,), jnp.int32),
        pltpu.VMEM((CHUNK,), jnp.int32),
        pltpu.VMEM((CHUNK,), jnp.int32),
        pltpu.VMEM((CHUNK, D), jnp.float32),
        pltpu.VMEM((CHUNK, D), jnp.float32),
        pltpu.VMEM((ZR, D), jnp.float32),
        pltpu.VMEM_SHARED((ACC_ROWS, D), jnp.float32),
        pltpu.SemaphoreType.DMA,
        pltpu.SemaphoreType.DMA,
    ],
)
def _edge_scatter(table_hbm, src_hbm, et_hbm, dst_hbm, out_hbm,
                  src_v, et_v, dst_v, g_a, g_b, d_a, d_b, rows_a, rows_b,
                  z_v, acc, sem_a, sem_b):
    cid = lax.axis_index("c")
    sid = lax.axis_index("s")
    wid = cid * NS + sid

    @pl.loop(0, ZR)
    def _(r):
        @pl.loop(0, D // L)
        def _(c):
            z_v.at[pl.ds(r, 1), pl.ds(c * L, L)][...] = jnp.zeros(
                (1, L), jnp.float32)

    @pl.loop(0, RPT // ZR)
    def _(j):
        pltpu.sync_copy(z_v, acc.at[pl.ds(sid * RPT + j * ZR, ZR)])

    base = wid * EPT
    pltpu.sync_copy(src_hbm.at[pl.ds(base, EPT)], src_v)
    pltpu.sync_copy(et_hbm.at[pl.ds(base, EPT)], et_v)
    pltpu.sync_copy(dst_hbm.at[pl.ds(base, EPT)], dst_v)

    def compute_idx(i, g, d):
        off = i * CHUNK

        @pl.loop(0, CHUNK // L)
        def _(j):
            s_in = pl.ds(off + j * L, L)
            s_out = pl.ds(j * L, L)
            g.at[s_out][...] = src_v.at[s_in][...] * 4 + et_v.at[s_in][...]
            d.at[s_out][...] = dst_v.at[s_in][...]

    compute_idx(0, g_a, d_a)
    pltpu.async_copy(table_hbm.at[g_a], rows_a, sem_a)

    plsc.subcore_barrier()

    @pl.loop(0, NCHUNK, step=2)
    def _(i):
        compute_idx(i + 1, g_b, d_b)
        pltpu.async_copy(table_hbm.at[g_b], rows_b, sem_b)

        pltpu.make_async_copy(table_hbm.at[g_a], rows_a, sem_a).wait()
        pltpu.sync_copy(rows_a, acc.at[d_a], add=True)

        @pl.when(i + 2 < NCHUNK)
        def _():
            compute_idx(i + 2, g_a, d_a)
            pltpu.async_copy(table_hbm.at[g_a], rows_a, sem_a)

        pltpu.make_async_copy(table_hbm.at[g_b], rows_b, sem_b).wait()
        pltpu.sync_copy(rows_b, acc.at[d_b], add=True)

    plsc.subcore_barrier()

    pltpu.sync_copy(acc.at[pl.ds(sid * RPT, RPT)],
                    out_hbm.at[cid, pl.ds(sid * RPT, RPT)])


def kernel(x, edge_index, edge_type, W0, W1, W2, Ws, bs):
    x = x.astype(jnp.float32)
    src = edge_index[0].astype(jnp.int32)
    dst = edge_index[1].astype(jnp.int32)
    et = edge_type.astype(jnp.int32)

    pad = E_PAD - N_EDGES
    src = jnp.pad(src, (0, pad))
    et = jnp.pad(et, (0, pad))
    dst = jnp.pad(dst, (0, pad), constant_values=N_NODES)

    w_cat = jnp.concatenate([W0, W1, W2, Ws], axis=0).T
    b_cat = jnp.zeros((1, 4 * D), jnp.float32).at[0, 3 * D:].set(bs)

    table = _transform(x, w_cat, b_cat)
    partials = _edge_scatter(table.reshape(4 * N_NODES, D), src, et, dst)
    return _combine(table, partials)
